# baseline (device time: 186036 ns/iter reference)
import jax
import jax.numpy as jnp
from jax import lax
from jax.experimental import pallas as pl
from jax.experimental.pallas import tpu as pltpu

N_DEV = 16
B, SQ, SKV, DH = 2, 512, 512, 64
HQ_LOCAL = 8
D_MODEL = 768
ROWS = B * SQ
CHUNK = ROWS // N_DEV


def _body(x_ref, wq_ref, k_ref, v_ref, wo_ref, out_ref,
          acc_ref, ctx_ref, rs_recv_ref,
          rs_send_sems, rs_recv_sems, ag_send_sems, ag_recv_sems):
    my = lax.axis_index("i")
    left = lax.rem(my - 1 + N_DEV, N_DEV)
    right = lax.rem(my + 1, N_DEV)

    barrier_sem = pltpu.get_barrier_semaphore()
    for nbr in (left, right):
        pl.semaphore_signal(barrier_sem, inc=1, device_id=(nbr,),
                            device_id_type=pl.DeviceIdType.MESH)
    pl.semaphore_wait(barrier_sem, 2)

    qb_i = lax.broadcasted_iota(jnp.int32, (SQ, SKV), 0) // 64
    kb_i = lax.broadcasted_iota(jnp.int32, (SQ, SKV), 1) // 64
    mask = (kb_i % 4) == (qb_i % 4)
    for b in range(B):
        q_b = jnp.dot(x_ref[b], wq_ref[...],
                      preferred_element_type=jnp.float32).astype(jnp.bfloat16)
        for h in range(HQ_LOCAL):
            q = q_b[:, h * DH:(h + 1) * DH]
            k = k_ref[b, h]
            v = v_ref[b, h]
            s = lax.dot_general(q, k, (((1,), (1,)), ((), ())),
                                preferred_element_type=jnp.float32) * 0.125
            s = jnp.where(mask, s, -1e9)
            m = jnp.max(s, axis=1, keepdims=True)
            w = jnp.exp(s - m)
            w = w / jnp.sum(w, axis=1, keepdims=True)
            ctx = jnp.dot(w.astype(jnp.bfloat16), v,
                          preferred_element_type=jnp.float32)
            ctx_ref[b, :, h * DH:(h + 1) * DH] = ctx.astype(jnp.bfloat16)
        acc_ref[b * SQ:(b + 1) * SQ, :] = jnp.dot(
            ctx_ref[b], wo_ref[...], preferred_element_type=jnp.float32)

    for s in range(N_DEV - 1):
        c_send = lax.rem(my - s + 2 * N_DEV, N_DEV)
        rdma = pltpu.make_async_remote_copy(
            src_ref=acc_ref.at[pl.ds(c_send * CHUNK, CHUNK), :],
            dst_ref=rs_recv_ref.at[s],
            send_sem=rs_send_sems.at[s],
            recv_sem=rs_recv_sems.at[s],
            device_id=(right,),
            device_id_type=pl.DeviceIdType.MESH,
        )
        rdma.start()
        rdma.wait()
        c_recv = lax.rem(my - s - 1 + 2 * N_DEV, N_DEV)
        cur = acc_ref[pl.ds(c_recv * CHUNK, CHUNK), :]
        acc_ref[pl.ds(c_recv * CHUNK, CHUNK), :] = cur + rs_recv_ref[s]

    for s in range(N_DEV - 1):
        c_send = lax.rem(my + 1 - s + 2 * N_DEV, N_DEV)
        rdma = pltpu.make_async_remote_copy(
            src_ref=acc_ref.at[pl.ds(c_send * CHUNK, CHUNK), :],
            dst_ref=acc_ref.at[pl.ds(c_send * CHUNK, CHUNK), :],
            send_sem=ag_send_sems.at[s],
            recv_sem=ag_recv_sems.at[s],
            device_id=(right,),
            device_id_type=pl.DeviceIdType.MESH,
        )
        rdma.start()
        rdma.wait()

    out_ref[0, :, :] = acc_ref[0:SQ, :]
    out_ref[1, :, :] = acc_ref[SQ:ROWS, :]


def kernel(x, Wq, K_ext, V_ext, Wo):
    i = lax.axis_index("i")
    k_sl = lax.dynamic_slice_in_dim(K_ext, i * HQ_LOCAL, HQ_LOCAL, axis=2)
    v_sl = lax.dynamic_slice_in_dim(V_ext, i * HQ_LOCAL, HQ_LOCAL, axis=2)
    k_sl = jnp.transpose(k_sl, (0, 2, 1, 3)).astype(jnp.bfloat16)
    v_sl = jnp.transpose(v_sl, (0, 2, 1, 3)).astype(jnp.bfloat16)

    return pl.pallas_call(
        _body,
        out_shape=jax.ShapeDtypeStruct((B, SQ, D_MODEL), jnp.float32),
        in_specs=[pl.BlockSpec(memory_space=pltpu.VMEM)] * 5,
        out_specs=pl.BlockSpec(memory_space=pltpu.VMEM),
        scratch_shapes=[
            pltpu.VMEM((ROWS, D_MODEL), jnp.float32),
            pltpu.VMEM((B, SQ, HQ_LOCAL * DH), jnp.bfloat16),
            pltpu.VMEM((N_DEV - 1, CHUNK, D_MODEL), jnp.float32),
            pltpu.SemaphoreType.DMA((N_DEV - 1,)),
            pltpu.SemaphoreType.DMA((N_DEV - 1,)),
            pltpu.SemaphoreType.DMA((N_DEV - 1,)),
            pltpu.SemaphoreType.DMA((N_DEV - 1,)),
        ],
        compiler_params=pltpu.CompilerParams(collective_id=0),
    )(x.astype(jnp.bfloat16), Wq.astype(jnp.bfloat16), k_sl, v_sl,
      Wo.astype(jnp.bfloat16))


# device time: 101956 ns/iter; 1.8247x vs baseline; 1.8247x over previous
import jax
import jax.numpy as jnp
from jax import lax
from jax.experimental import pallas as pl
from jax.experimental.pallas import tpu as pltpu

N_DEV = 16
B, SQ, SKV, DH = 2, 512, 512, 64
HQ_LOCAL = 8
D_MODEL = 768
ROWS = B * SQ
CHUNK = ROWS // N_DEV

_MESH = pl.DeviceIdType.MESH


def _body(x_ref, wq_ref, k_ref, v_ref, wo_ref, out_ref,
          snd_ref, ctx_ref, a2a_ref, red_ref, flat_ref,
          s1_send, s1_recv, s2_send, s2_recv):
    my = lax.axis_index("i")

    barrier_sem = pltpu.get_barrier_semaphore()
    for dj in range(1, N_DEV):
        peer = lax.rem(my + dj, N_DEV)
        pl.semaphore_signal(barrier_sem, inc=1, device_id=(peer,),
                            device_id_type=_MESH)
    pl.semaphore_wait(barrier_sem, N_DEV - 1)

    qb_i = lax.broadcasted_iota(jnp.int32, (SQ, SKV), 0) // 64
    kb_i = lax.broadcasted_iota(jnp.int32, (SQ, SKV), 1) // 64
    mask = (kb_i % 4) == (qb_i % 4)
    for b in range(B):
        q_b = jnp.dot(x_ref[b], wq_ref[...],
                      preferred_element_type=jnp.float32).astype(jnp.bfloat16)
        for h in range(HQ_LOCAL):
            q = q_b[:, h * DH:(h + 1) * DH]
            s = lax.dot_general(q, k_ref[b, h], (((1,), (1,)), ((), ())),
                                preferred_element_type=jnp.float32) * 0.125
            s = jnp.where(mask, s, -1e9)
            m = jnp.max(s, axis=1, keepdims=True)
            w = jnp.exp(s - m)
            w = w / jnp.sum(w, axis=1, keepdims=True)
            ctx = jnp.dot(w.astype(jnp.bfloat16), v_ref[b, h],
                          preferred_element_type=jnp.float32)
            ctx_ref[b, :, h * DH:(h + 1) * DH] = ctx.astype(jnp.bfloat16)
        proj = jnp.dot(ctx_ref[b], wo_ref[...],
                       preferred_element_type=jnp.float32)
        snd_ref[b * SQ:(b + 1) * SQ, :] = proj.astype(jnp.bfloat16)

    sends1 = []
    for dj in range(1, N_DEV):
        d = lax.rem(my + dj, N_DEV)
        rdma = pltpu.make_async_remote_copy(
            src_ref=snd_ref.at[pl.ds(d * CHUNK, CHUNK), :],
            dst_ref=a2a_ref.at[dj - 1],
            send_sem=s1_send.at[dj - 1],
            recv_sem=s1_recv.at[dj - 1],
            device_id=(d,),
            device_id_type=_MESH,
        )
        rdma.start()
        sends1.append(rdma)

    red = snd_ref[pl.ds(my * CHUNK, CHUNK), :].astype(jnp.float32)
    for k in range(N_DEV - 1):
        recv = pltpu.make_async_remote_copy(
            src_ref=a2a_ref.at[k], dst_ref=a2a_ref.at[k],
            send_sem=s1_send.at[k], recv_sem=s1_recv.at[k],
            device_id=(my,), device_id_type=_MESH,
        )
        recv.wait_recv()
        red = red + a2a_ref[k].astype(jnp.float32)
    red_ref[...] = red.astype(jnp.bfloat16)
    flat_ref[pl.ds(my * CHUNK, CHUNK), :] = red_ref[...]
    for r in sends1:
        r.wait_send()

    sends2 = []
    for dj in range(1, N_DEV):
        d = lax.rem(my + dj, N_DEV)
        rdma = pltpu.make_async_remote_copy(
            src_ref=red_ref,
            dst_ref=flat_ref.at[pl.ds(my * CHUNK, CHUNK), :],
            send_sem=s2_send.at[dj - 1],
            recv_sem=s2_recv.at[dj - 1],
            device_id=(d,),
            device_id_type=_MESH,
        )
        rdma.start()
        sends2.append(rdma)

    for k in range(N_DEV - 1):
        recv = pltpu.make_async_remote_copy(
            src_ref=red_ref, dst_ref=red_ref,
            send_sem=s2_send.at[k], recv_sem=s2_recv.at[k],
            device_id=(my,), device_id_type=_MESH,
        )
        recv.wait_recv()
    for r in sends2:
        r.wait_send()

    out_ref[0, :, :] = flat_ref[0:SQ, :].astype(jnp.float32)
    out_ref[1, :, :] = flat_ref[SQ:ROWS, :].astype(jnp.float32)


def kernel(x, Wq, K_ext, V_ext, Wo):
    i = lax.axis_index("i")
    k_sl = lax.dynamic_slice_in_dim(K_ext, i * HQ_LOCAL, HQ_LOCAL, axis=2)
    v_sl = lax.dynamic_slice_in_dim(V_ext, i * HQ_LOCAL, HQ_LOCAL, axis=2)
    k_sl = jnp.transpose(k_sl, (0, 2, 1, 3)).astype(jnp.bfloat16)
    v_sl = jnp.transpose(v_sl, (0, 2, 1, 3)).astype(jnp.bfloat16)

    return pl.pallas_call(
        _body,
        out_shape=jax.ShapeDtypeStruct((B, SQ, D_MODEL), jnp.float32),
        in_specs=[pl.BlockSpec(memory_space=pltpu.VMEM)] * 5,
        out_specs=pl.BlockSpec(memory_space=pltpu.VMEM),
        scratch_shapes=[
            pltpu.VMEM((ROWS, D_MODEL), jnp.bfloat16),
            pltpu.VMEM((B, SQ, HQ_LOCAL * DH), jnp.bfloat16),
            pltpu.VMEM((N_DEV - 1, CHUNK, D_MODEL), jnp.bfloat16),
            pltpu.VMEM((CHUNK, D_MODEL), jnp.bfloat16),
            pltpu.VMEM((ROWS, D_MODEL), jnp.bfloat16),
            pltpu.SemaphoreType.DMA((N_DEV - 1,)),
            pltpu.SemaphoreType.DMA((N_DEV - 1,)),
            pltpu.SemaphoreType.DMA((N_DEV - 1,)),
            pltpu.SemaphoreType.DMA((N_DEV - 1,)),
        ],
        compiler_params=pltpu.CompilerParams(collective_id=0),
    )(x.astype(jnp.bfloat16), Wq.astype(jnp.bfloat16), k_sl, v_sl,
      Wo.astype(jnp.bfloat16))
